# baseline (speedup 1.0000x reference)
import functools

import jax
import jax.numpy as jnp
from jax import lax
from jax.experimental import pallas as pl
from jax.experimental.pallas import tpu as pltpu
from jax.experimental.pallas import tpu_sc as plsc

V = 10000
E = 320000
D = 128

_VB = 2000
_EB = 3200
_NW = 32
_C = 80
_NP = 1
_EC = E // _NP
_EWC = _EC // _NW
_NCHC = _EWC // _C
_VP = 10240


def _sigmoid(v):
    return 0.5 * jnp.tanh(0.5 * v) + 0.5


def _silu(v):
    return v * _sigmoid(v)


def _pack_bf16_pair(v):
    u = jax.lax.bitcast_convert_type(v, jnp.uint32)
    r = u + jnp.uint32(0x7FFF) + ((u >> 16) & jnp.uint32(1))
    hi = r[:, 0:64] & jnp.uint32(0xFFFF0000)
    lo = r[:, 64:128] >> 16
    return jax.lax.bitcast_convert_type(hi | lo, jnp.float32)


def _unpack_bf16_pair(p):
    u = jax.lax.bitcast_convert_type(p, jnp.uint32)
    hi = jax.lax.bitcast_convert_type(u & jnp.uint32(0xFFFF0000), jnp.float32)
    lo = jax.lax.bitcast_convert_type(u << 16, jnp.float32)
    return jnp.concatenate([hi, lo], axis=1)


def _prep_body(h_ref, w1a_ref, w1b_ref, a_ref, b_ref):
    hb = h_ref[...]
    a_ref[...] = _pack_bf16_pair(
        jnp.dot(hb, w1a_ref[...], preferred_element_type=jnp.float32))
    b_ref[...] = _pack_bf16_pair(
        jnp.dot(hb, w1b_ref[...], preferred_element_type=jnp.float32))


def _prep_call(h, w1a, w1b):
    return pl.pallas_call(
        _prep_body,
        grid=(V // _VB,),
        in_specs=[
            pl.BlockSpec((_VB, D), lambda i: (i, 0)),
            pl.BlockSpec((D, D), lambda i: (0, 0)),
            pl.BlockSpec((D, D), lambda i: (0, 0)),
        ],
        out_specs=[
            pl.BlockSpec((_VB, 64), lambda i: (i, 0)),
            pl.BlockSpec((_VB, 64), lambda i: (i, 0)),
        ],
        out_shape=[
            jax.ShapeDtypeStruct((V, 64), jnp.float32),
            jax.ShapeDtypeStruct((V, 64), jnp.float32),
        ],
    )(h, w1a, w1b)


def _sc_gather(a_tab, b_tab, p_tab, row, col, e0):
    mesh = plsc.VectorSubcoreMesh(core_axis_name="c", subcore_axis_name="s")

    @functools.partial(
        pl.kernel,
        mesh=mesh,
        compiler_params=pltpu.CompilerParams(use_tc_tiling_on_sc=False),
        out_type=[
            jax.ShapeDtypeStruct((_EC, 64), jnp.float32),
            jax.ShapeDtypeStruct((_EC, 64), jnp.float32),
            jax.ShapeDtypeStruct((_EC, 16), jnp.float32),
            jax.ShapeDtypeStruct((_EC, 16), jnp.float32),
        ],
        scratch_types=[
            pltpu.VMEM((_C,), jnp.int32),
            pltpu.VMEM((_C,), jnp.int32),
            pltpu.VMEM((_C, 64), jnp.float32),
            pltpu.VMEM((_C, 64), jnp.float32),
            pltpu.VMEM((_C, 16), jnp.float32),
            pltpu.VMEM((_C, 16), jnp.float32),
            pltpu.SemaphoreType.DMA,
            pltpu.SemaphoreType.DMA,
            pltpu.SemaphoreType.DMA,
            pltpu.SemaphoreType.DMA,
        ],
    )
    def k(a_hbm, b_hbm, p_hbm, row_hbm, col_hbm,
          ar_hbm, bc_hbm, pr_hbm, pc_hbm,
          idxr, idxc, bufa, bufb, bufpr, bufpc, sema, semb, semc, semd):
        w = lax.axis_index("s") * 2 + lax.axis_index("c")

        def body(i, carry):
            loc = w * _EWC + i * _C
            gbase = e0 + loc
            pltpu.sync_copy(row_hbm.at[pl.ds(gbase, _C)], idxr)
            pltpu.sync_copy(col_hbm.at[pl.ds(gbase, _C)], idxc)
            ca = pltpu.async_copy(a_hbm.at[idxr], bufa, sema)
            cb = pltpu.async_copy(b_hbm.at[idxc], bufb, semb)
            cpr = pltpu.async_copy(p_hbm.at[idxr], bufpr, semc)
            cpc = pltpu.async_copy(p_hbm.at[idxc], bufpc, semd)
            ca.wait()
            cb.wait()
            cpr.wait()
            cpc.wait()
            pltpu.sync_copy(bufa, ar_hbm.at[pl.ds(loc, _C)])
            pltpu.sync_copy(bufb, bc_hbm.at[pl.ds(loc, _C)])
            pltpu.sync_copy(bufpr, pr_hbm.at[pl.ds(loc, _C)])
            pltpu.sync_copy(bufpc, pc_hbm.at[pl.ds(loc, _C)])
            return carry

        lax.fori_loop(0, _NCHC, body, 0)

    return k(a_tab, b_tab, p_tab, row, col)


def _edge_body(ar_ref, bc_ref, pr_ref, pc_ref,
               w1l_ref, be1_ref, we2_ref, be2_ref,
               wh_ref, bh_ref, wcb2_ref, wc12_ref, wc22_ref,
               wa1_ref, w16_ref, ba1_ref, wa2_ref, ba2_ref,
               mw_ref, cd_ref):
    pr = pr_ref[...]
    pc = pc_ref[...]
    xr = pr[:, 0:3]
    pd1r = pr[:, 4:7]
    pd2r = pr[:, 8:11]
    curv = pr[:, 12:16]
    xc = pc[:, 0:3]
    diff = xc - xr
    d2 = jnp.sum(diff * diff, axis=1, keepdims=True)
    dist = jnp.maximum(jnp.sqrt(d2), 1e-6)
    inv_dist = 1.0 / dist
    e = diff * inv_dist
    c1 = jnp.sum(e * pd1r, axis=1, keepdims=True)
    c2 = jnp.sum(e * pd2r, axis=1, keepdims=True)

    msk = jnp.uint32(0xFFFF0000)
    a_u = jax.lax.bitcast_convert_type(ar_ref[...], jnp.uint32)
    b_u = jax.lax.bitcast_convert_type(bc_ref[...], jnp.uint32)
    f32 = lambda q: jax.lax.bitcast_convert_type(q, jnp.float32)
    t1h = (f32(a_u & msk) + f32(b_u & msk)
           + d2 * w1l_ref[:, 0:64] + be1_ref[:, 0:64])
    t1l = (f32(a_u << 16) + f32(b_u << 16)
           + d2 * w1l_ref[:, 64:128] + be1_ref[:, 64:128])
    uh = _silu(t1h)
    ul = _silu(t1l)
    bf = jnp.bfloat16
    m_ij = _silu(jnp.dot(uh.astype(bf), we2_ref[0:64].astype(bf),
                         preferred_element_type=jnp.float32)
                 + jnp.dot(ul.astype(bf), we2_ref[64:128].astype(bf),
                           preferred_element_type=jnp.float32)
                 + be2_ref[...])

    k1 = curv[:, 0:1]
    k2 = curv[:, 1:2]
    hid = (jnp.dot(pr, w16_ref[...], preferred_element_type=jnp.float32)
           + ba1_ref[...])
    terms = (jnp.abs(c1), jnp.abs(c2), c1 * k1, c2 * k2, dist)
    for kk, t in zip((0, 1, 2, 3, 8), terms):
        hid = hid + t * wa1_ref[kk:kk + 1, :]
    hid = _silu(hid)
    w_an = _sigmoid(jnp.sum(hid * wa2_ref[...], axis=1, keepdims=True)
                    + ba2_ref[...]) + 0.5

    m_w = m_ij * w_an
    m_wb = m_w.astype(bf)

    ch = _silu(jnp.dot(m_wb, wh_ref[...], preferred_element_type=jnp.float32)
               + bh_ref[...])
    alpha_b = jnp.tanh(jnp.sum(ch[:, 0:128] * wcb2_ref[...],
                               axis=1, keepdims=True))
    alpha1 = jnp.tanh(jnp.sum(ch[:, 128:192] * wc12_ref[...],
                              axis=1, keepdims=True))
    alpha2 = jnp.tanh(jnp.sum(ch[:, 192:256] * wc22_ref[...],
                              axis=1, keepdims=True))

    inv_dn = 1.0 / (dist + 1.0)
    cd = ((alpha_b * inv_dn) * diff + (alpha1 * c1) * pd1r
          + (alpha2 * c2) * pd2r)

    mw_ref[...] = m_w
    cd_ref[...] = jnp.concatenate(
        [cd, jnp.zeros((cd.shape[0], 5), jnp.float32)], axis=1)


def _edge_call(ar, bc, prg, pcg, w1l, be1, we2, be2, wh, bh,
               wcb2r, wc12r, wc22r, wa1, w16, ba1, wa2r, ba2):
    full = lambda shape: pl.BlockSpec(shape, lambda i: tuple(0 for _ in shape))
    return pl.pallas_call(
        _edge_body,
        grid=(_EC // _EB,),
        in_specs=[
            pl.BlockSpec((_EB, 64), lambda i: (i, 0)),
            pl.BlockSpec((_EB, 64), lambda i: (i, 0)),
            pl.BlockSpec((_EB, 16), lambda i: (i, 0)),
            pl.BlockSpec((_EB, 16), lambda i: (i, 0)),
            full((1, D)), full((1, D)), full((D, D)), full((1, D)),
            full((D, 256)), full((1, 256)),
            full((1, D)), full((1, 64)), full((1, 64)),
            full((9, 32)), full((16, 32)), full((1, 32)), full((1, 32)),
            full((1, 1)),
        ],
        out_specs=[
            pl.BlockSpec((_EB, D), lambda i: (i, 0)),
            pl.BlockSpec((_EB, 8), lambda i: (i, 0)),
        ],
        out_shape=[
            jax.ShapeDtypeStruct((_EC, D), jnp.float32),
            jax.ShapeDtypeStruct((_EC, 8), jnp.float32),
        ],
    )(ar, bc, prg, pcg, w1l, be1, we2, be2, wh, bh,
      wcb2r, wc12r, wc22r, wa1, w16, ba1, wa2r, ba2)


def _sc_scatter(mws, cds, col, z128, z8):
    mesh = plsc.VectorSubcoreMesh(core_axis_name="c", subcore_axis_name="s")
    rows = _VP // 16

    @functools.partial(
        pl.kernel,
        mesh=mesh,
        compiler_params=pltpu.CompilerParams(use_tc_tiling_on_sc=False),
        out_type=[
            jax.ShapeDtypeStruct((2 * _VP, D), jnp.float32),
            jax.ShapeDtypeStruct((2 * _VP, 8), jnp.float32),
        ],
        scratch_types=[
            pltpu.VMEM((_C,), jnp.int32),
            pltpu.VMEM((_C, D), jnp.float32),
            pltpu.VMEM((_C, 8), jnp.float32),
            pltpu.VMEM_SHARED((_VP, D), jnp.float32),
            pltpu.VMEM_SHARED((_VP, 8), jnp.float32),
            pltpu.SemaphoreType.DMA,
            pltpu.SemaphoreType.DMA,
        ],
    )
    def k(mw0, cd0,
          col_hbm, z128_hbm, z8_hbm, out1_hbm, out2_hbm,
          idxc, bufm, bufc, acc128, acc8, sem1, sem2):
        mw_hbms = (mw0,)
        cd_hbms = (cd0,)
        c = lax.axis_index("c")
        s = lax.axis_index("s")
        w = s * 2 + c

        pltpu.sync_copy(z128_hbm.at[pl.ds(s * rows, rows)],
                        acc128.at[pl.ds(s * rows, rows)])
        pltpu.sync_copy(z8_hbm.at[pl.ds(s * rows, rows)],
                        acc8.at[pl.ds(s * rows, rows)])
        plsc.subcore_barrier()

        for ci in range(_NP):
            mw_hbm = mw_hbms[ci]
            cd_hbm = cd_hbms[ci]

            def body(i, carry):
                loc = w * _EWC + i * _C
                pltpu.sync_copy(col_hbm.at[pl.ds(ci * _EC + loc, _C)], idxc)
                cm = pltpu.async_copy(mw_hbm.at[pl.ds(loc, _C)], bufm, sem1)
                cc = pltpu.async_copy(cd_hbm.at[pl.ds(loc, _C)], bufc, sem2)
                cm.wait()
                cc.wait()
                pltpu.sync_copy(bufm, acc128.at[idxc], add=True)
                pltpu.sync_copy(bufc, acc8.at[idxc], add=True)
                return carry

            lax.fori_loop(0, _NCHC, body, 0)
        plsc.subcore_barrier()

        pltpu.sync_copy(acc128.at[pl.ds(s * rows, rows)],
                        out1_hbm.at[pl.ds(c * _VP + s * rows, rows)])
        pltpu.sync_copy(acc8.at[pl.ds(s * rows, rows)],
                        out2_hbm.at[pl.ds(c * _VP + s * rows, rows)])

    return k(*mws, *cds, col, z128, z8)


def _node_body(h_ref, x_ref, s0_ref, s1_ref, c0_ref, c1_ref,
               wn1a_ref, wn1b_ref, bn1_ref, wn2_ref, bn2_ref,
               g_ref, b_ref, hout_ref, xout_ref):
    hb = h_ref[...]
    msg = s0_ref[...] + s1_ref[...]
    u = _silu(jnp.dot(hb, wn1a_ref[...], preferred_element_type=jnp.float32)
              + jnp.dot(msg, wn1b_ref[...], preferred_element_type=jnp.float32)
              + bn1_ref[...])
    hupd = jnp.dot(u, wn2_ref[...], preferred_element_type=jnp.float32) + bn2_ref[...]
    hh = hb + hupd
    mu = jnp.mean(hh, axis=1, keepdims=True)
    var = jnp.mean((hh - mu) ** 2, axis=1, keepdims=True)
    hout_ref[...] = (hh - mu) * lax.rsqrt(var + 1e-5) * g_ref[...] + b_ref[...]
    xout_ref[...] = x_ref[...] + c0_ref[:, 0:3] + c1_ref[:, 0:3]


def _node_call(h, x, s0, s1, c0, c1, wn1a, wn1b, bn1, wn2, bn2, g_ln, b_ln):
    full = lambda shape: pl.BlockSpec(shape, lambda i: tuple(0 for _ in shape))
    return pl.pallas_call(
        _node_body,
        grid=(V // _VB,),
        in_specs=[
            pl.BlockSpec((_VB, D), lambda i: (i, 0)),
            pl.BlockSpec((_VB, 3), lambda i: (i, 0)),
            pl.BlockSpec((_VB, D), lambda i: (i, 0)),
            pl.BlockSpec((_VB, D), lambda i: (i, 0)),
            pl.BlockSpec((_VB, 8), lambda i: (i, 0)),
            pl.BlockSpec((_VB, 8), lambda i: (i, 0)),
            full((D, D)), full((D, D)), full((1, D)),
            full((D, D)), full((1, D)), full((1, D)), full((1, D)),
        ],
        out_specs=[
            pl.BlockSpec((_VB, D), lambda i: (i, 0)),
            pl.BlockSpec((_VB, 3), lambda i: (i, 0)),
        ],
        out_shape=[
            jax.ShapeDtypeStruct((V, D), jnp.float32),
            jax.ShapeDtypeStruct((V, 3), jnp.float32),
        ],
    )(h, x, s0, s1, c0, c1, wn1a, wn1b, bn1, wn2, bn2, g_ln, b_ln)


def kernel(h, x, edge_index, curvature, pd1, pd2, We1, be1, We2, be2,
           Wcb1, bcb1, Wcb2, Wc11, bc11, Wc12, Wc21, bc21, Wc22,
           Wn1, bn1, Wn2, bn2, g_ln, b_ln, Wa1, ba1, Wa2, ba2):
    row = edge_index[0]
    col = edge_index[1]

    w1a = We1[0:D]
    w1b = We1[D:2 * D]
    w1l = We1[2 * D].reshape(1, D)

    pad1 = jnp.zeros((V, 1), jnp.float32)
    P = jnp.concatenate([x, pad1, pd1, pad1, pd2, pad1, curvature], axis=1)

    A, B = _prep_call(h, w1a, w1b)

    wh = jnp.concatenate([Wcb1, Wc11, Wc21], axis=1).astype(jnp.bfloat16)
    bh = jnp.concatenate([bcb1, bc11, bc21]).reshape(1, 256)
    w16 = jnp.zeros((16, 32), jnp.float32).at[12:16].set(Wa1[4:8])

    mws, cds = [], []
    for ci in range(_NP):
        ar, bc, prg, pcg = _sc_gather(A, B, P, row, col, ci * _EC)
        mw, cd8 = _edge_call(
            ar, bc, prg, pcg,
            w1l, be1.reshape(1, D), We2, be2.reshape(1, D), wh, bh,
            Wcb2.reshape(1, D), Wc12.reshape(1, 64), Wc22.reshape(1, 64),
            Wa1, w16, ba1.reshape(1, 32), Wa2.reshape(1, 32),
            ba2.reshape(1, 1))
        mws.append(mw)
        cds.append(cd8)

    z128 = jnp.zeros((_VP, D), jnp.float32)
    z8 = jnp.zeros((_VP, 8), jnp.float32)
    s_part, c_part = _sc_scatter(mws, cds, col, z128, z8)

    h_out, x_new = _node_call(
        h, x, s_part[:V], s_part[_VP:_VP + V],
        c_part[:V], c_part[_VP:_VP + V],
        Wn1[0:D], Wn1[D:2 * D], bn1.reshape(1, D), Wn2, bn2.reshape(1, D),
        g_ln.reshape(1, D), b_ln.reshape(1, D))
    return (h_out, x_new)


# speedup vs baseline: 1.0010x; 1.0010x over previous
import functools

import jax
import jax.numpy as jnp
from jax import lax
from jax.experimental import pallas as pl
from jax.experimental.pallas import tpu as pltpu
from jax.experimental.pallas import tpu_sc as plsc

V = 10000
E = 320000
D = 128

_VB = 2000
_EB = 3200
_NW = 32
_C = 80
_NP = 1
_EC = E // _NP
_EWC = _EC // _NW
_NCHC = _EWC // _C
_VP = 10240


def _sigmoid(v):
    return 0.5 * jnp.tanh(0.5 * v) + 0.5


def _silu(v):
    return v * _sigmoid(v)


def _pack_bf16_pair(v):
    u = jax.lax.bitcast_convert_type(v, jnp.uint32)
    r = u + jnp.uint32(0x7FFF) + ((u >> 16) & jnp.uint32(1))
    hi = r[:, 0:64] & jnp.uint32(0xFFFF0000)
    lo = r[:, 64:128] >> 16
    return jax.lax.bitcast_convert_type(hi | lo, jnp.float32)


def _unpack_bf16_pair(p):
    u = jax.lax.bitcast_convert_type(p, jnp.uint32)
    hi = jax.lax.bitcast_convert_type(u & jnp.uint32(0xFFFF0000), jnp.float32)
    lo = jax.lax.bitcast_convert_type(u << 16, jnp.float32)
    return jnp.concatenate([hi, lo], axis=1)


def _prep_body(h_ref, w1a_ref, w1b_ref, a_ref, b_ref):
    hb = h_ref[...]
    a_ref[...] = _pack_bf16_pair(
        jnp.dot(hb, w1a_ref[...], preferred_element_type=jnp.float32))
    b_ref[...] = _pack_bf16_pair(
        jnp.dot(hb, w1b_ref[...], preferred_element_type=jnp.float32))


def _prep_call(h, w1a, w1b):
    return pl.pallas_call(
        _prep_body,
        grid=(V // _VB,),
        in_specs=[
            pl.BlockSpec((_VB, D), lambda i: (i, 0)),
            pl.BlockSpec((D, D), lambda i: (0, 0)),
            pl.BlockSpec((D, D), lambda i: (0, 0)),
        ],
        out_specs=[
            pl.BlockSpec((_VB, 64), lambda i: (i, 0)),
            pl.BlockSpec((_VB, 64), lambda i: (i, 0)),
        ],
        out_shape=[
            jax.ShapeDtypeStruct((V, 64), jnp.float32),
            jax.ShapeDtypeStruct((V, 64), jnp.float32),
        ],
    )(h, w1a, w1b)


def _sc_gather(a_tab, b_tab, p_tab, row, col, e0):
    mesh = plsc.VectorSubcoreMesh(core_axis_name="c", subcore_axis_name="s")

    @functools.partial(
        pl.kernel,
        mesh=mesh,
        compiler_params=pltpu.CompilerParams(use_tc_tiling_on_sc=False),
        out_type=[
            jax.ShapeDtypeStruct((_EC, 64), jnp.float32),
            jax.ShapeDtypeStruct((_EC, 64), jnp.float32),
            jax.ShapeDtypeStruct((_EC, 16), jnp.float32),
            jax.ShapeDtypeStruct((_EC, 16), jnp.float32),
        ],
        scratch_types=[
            pltpu.VMEM((_C,), jnp.int32),
            pltpu.VMEM((_C,), jnp.int32),
            pltpu.VMEM((_C, 64), jnp.float32),
            pltpu.VMEM((_C, 64), jnp.float32),
            pltpu.VMEM((_C, 16), jnp.float32),
            pltpu.VMEM((_C, 16), jnp.float32),
            pltpu.SemaphoreType.DMA,
            pltpu.SemaphoreType.DMA,
            pltpu.SemaphoreType.DMA,
            pltpu.SemaphoreType.DMA,
        ],
    )
    def k(a_hbm, b_hbm, p_hbm, row_hbm, col_hbm,
          ar_hbm, bc_hbm, pr_hbm, pc_hbm,
          idxr, idxc, bufa, bufb, bufpr, bufpc, sema, semb, semc, semd):
        w = lax.axis_index("s") * 2 + lax.axis_index("c")

        def body(i, carry):
            loc = w * _EWC + i * _C
            gbase = e0 + loc
            pltpu.sync_copy(row_hbm.at[pl.ds(gbase, _C)], idxr)
            pltpu.sync_copy(col_hbm.at[pl.ds(gbase, _C)], idxc)
            ca = pltpu.async_copy(a_hbm.at[idxr], bufa, sema)
            cb = pltpu.async_copy(b_hbm.at[idxc], bufb, semb)
            cpr = pltpu.async_copy(p_hbm.at[idxr], bufpr, semc)
            cpc = pltpu.async_copy(p_hbm.at[idxc], bufpc, semd)
            ca.wait()
            cb.wait()
            cpr.wait()
            cpc.wait()
            pltpu.sync_copy(bufa, ar_hbm.at[pl.ds(loc, _C)])
            pltpu.sync_copy(bufb, bc_hbm.at[pl.ds(loc, _C)])
            pltpu.sync_copy(bufpr, pr_hbm.at[pl.ds(loc, _C)])
            pltpu.sync_copy(bufpc, pc_hbm.at[pl.ds(loc, _C)])
            return carry

        lax.fori_loop(0, _NCHC, body, 0)

    return k(a_tab, b_tab, p_tab, row, col)


def _edge_body(ar_ref, bc_ref, pr_ref, pc_ref,
               w1l_ref, be1_ref, we2_ref, be2_ref,
               wh_ref, bh_ref, wcb2_ref, wc12_ref, wc22_ref,
               wa1_ref, w16_ref, ba1_ref, wa2_ref, ba2_ref,
               mw_ref, cd_ref):
    pr = pr_ref[...]
    pc = pc_ref[...]
    xr = pr[:, 0:3]
    pd1r = pr[:, 4:7]
    pd2r = pr[:, 8:11]
    curv = pr[:, 12:16]
    xc = pc[:, 0:3]
    diff = xc - xr
    d2 = jnp.sum(diff * diff, axis=1, keepdims=True)
    dist = jnp.maximum(jnp.sqrt(d2), 1e-6)
    inv_dist = 1.0 / dist
    e = diff * inv_dist
    c1 = jnp.sum(e * pd1r, axis=1, keepdims=True)
    c2 = jnp.sum(e * pd2r, axis=1, keepdims=True)

    msk = jnp.uint32(0xFFFF0000)
    a_u = jax.lax.bitcast_convert_type(ar_ref[...], jnp.uint32)
    b_u = jax.lax.bitcast_convert_type(bc_ref[...], jnp.uint32)
    f32 = lambda q: jax.lax.bitcast_convert_type(q, jnp.float32)
    t1h = (f32(a_u & msk) + f32(b_u & msk)
           + d2 * w1l_ref[:, 0:64] + be1_ref[:, 0:64])
    t1l = (f32(a_u << 16) + f32(b_u << 16)
           + d2 * w1l_ref[:, 64:128] + be1_ref[:, 64:128])
    uh = _silu(t1h)
    ul = _silu(t1l)
    m_ij = _silu(jnp.dot(uh, we2_ref[0:64], preferred_element_type=jnp.float32)
                 + jnp.dot(ul, we2_ref[64:128], preferred_element_type=jnp.float32)
                 + be2_ref[...])

    k1 = curv[:, 0:1]
    k2 = curv[:, 1:2]
    hid = (jnp.dot(pr, w16_ref[...], preferred_element_type=jnp.float32)
           + ba1_ref[...])
    terms = (jnp.abs(c1), jnp.abs(c2), c1 * k1, c2 * k2, dist)
    for kk, t in zip((0, 1, 2, 3, 8), terms):
        hid = hid + t * wa1_ref[kk:kk + 1, :]
    hid = _silu(hid)
    w_an = _sigmoid(jnp.sum(hid * wa2_ref[...], axis=1, keepdims=True)
                    + ba2_ref[...]) + 0.5

    m_w = m_ij * w_an

    ch = _silu(jnp.dot(m_w, wh_ref[...], preferred_element_type=jnp.float32)
               + bh_ref[...])
    alpha_b = jnp.tanh(jnp.sum(ch[:, 0:128] * wcb2_ref[...],
                               axis=1, keepdims=True))
    alpha1 = jnp.tanh(jnp.sum(ch[:, 128:192] * wc12_ref[...],
                              axis=1, keepdims=True))
    alpha2 = jnp.tanh(jnp.sum(ch[:, 192:256] * wc22_ref[...],
                              axis=1, keepdims=True))

    inv_dn = 1.0 / (dist + 1.0)
    cd = ((alpha_b * inv_dn) * diff + (alpha1 * c1) * pd1r
          + (alpha2 * c2) * pd2r)

    mw_ref[...] = m_w
    cd_ref[...] = jnp.concatenate(
        [cd, jnp.zeros((cd.shape[0], 5), jnp.float32)], axis=1)


def _edge_call(ar, bc, prg, pcg, w1l, be1, we2, be2, wh, bh,
               wcb2r, wc12r, wc22r, wa1, w16, ba1, wa2r, ba2):
    full = lambda shape: pl.BlockSpec(shape, lambda i: tuple(0 for _ in shape))
    return pl.pallas_call(
        _edge_body,
        grid=(_EC // _EB,),
        in_specs=[
            pl.BlockSpec((_EB, 64), lambda i: (i, 0)),
            pl.BlockSpec((_EB, 64), lambda i: (i, 0)),
            pl.BlockSpec((_EB, 16), lambda i: (i, 0)),
            pl.BlockSpec((_EB, 16), lambda i: (i, 0)),
            full((1, D)), full((1, D)), full((D, D)), full((1, D)),
            full((D, 256)), full((1, 256)),
            full((1, D)), full((1, 64)), full((1, 64)),
            full((9, 32)), full((16, 32)), full((1, 32)), full((1, 32)),
            full((1, 1)),
        ],
        out_specs=[
            pl.BlockSpec((_EB, D), lambda i: (i, 0)),
            pl.BlockSpec((_EB, 8), lambda i: (i, 0)),
        ],
        out_shape=[
            jax.ShapeDtypeStruct((_EC, D), jnp.float32),
            jax.ShapeDtypeStruct((_EC, 8), jnp.float32),
        ],
    )(ar, bc, prg, pcg, w1l, be1, we2, be2, wh, bh,
      wcb2r, wc12r, wc22r, wa1, w16, ba1, wa2r, ba2)


def _sc_scatter(mws, cds, col, z128, z8):
    mesh = plsc.VectorSubcoreMesh(core_axis_name="c", subcore_axis_name="s")
    rows = _VP // 16

    @functools.partial(
        pl.kernel,
        mesh=mesh,
        compiler_params=pltpu.CompilerParams(use_tc_tiling_on_sc=False),
        out_type=[
            jax.ShapeDtypeStruct((2 * _VP, D), jnp.float32),
            jax.ShapeDtypeStruct((2 * _VP, 8), jnp.float32),
        ],
        scratch_types=[
            pltpu.VMEM((_C,), jnp.int32),
            pltpu.VMEM((_C, D), jnp.float32),
            pltpu.VMEM((_C, 8), jnp.float32),
            pltpu.VMEM_SHARED((_VP, D), jnp.float32),
            pltpu.VMEM_SHARED((_VP, 8), jnp.float32),
            pltpu.SemaphoreType.DMA,
            pltpu.SemaphoreType.DMA,
        ],
    )
    def k(mw0, cd0,
          col_hbm, z128_hbm, z8_hbm, out1_hbm, out2_hbm,
          idxc, bufm, bufc, acc128, acc8, sem1, sem2):
        mw_hbms = (mw0,)
        cd_hbms = (cd0,)
        c = lax.axis_index("c")
        s = lax.axis_index("s")
        w = s * 2 + c

        pltpu.sync_copy(z128_hbm.at[pl.ds(s * rows, rows)],
                        acc128.at[pl.ds(s * rows, rows)])
        pltpu.sync_copy(z8_hbm.at[pl.ds(s * rows, rows)],
                        acc8.at[pl.ds(s * rows, rows)])
        plsc.subcore_barrier()

        for ci in range(_NP):
            mw_hbm = mw_hbms[ci]
            cd_hbm = cd_hbms[ci]

            def body(i, carry):
                loc = w * _EWC + i * _C
                pltpu.sync_copy(col_hbm.at[pl.ds(ci * _EC + loc, _C)], idxc)
                cm = pltpu.async_copy(mw_hbm.at[pl.ds(loc, _C)], bufm, sem1)
                cc = pltpu.async_copy(cd_hbm.at[pl.ds(loc, _C)], bufc, sem2)
                cm.wait()
                cc.wait()
                pltpu.sync_copy(bufm, acc128.at[idxc], add=True)
                pltpu.sync_copy(bufc, acc8.at[idxc], add=True)
                return carry

            lax.fori_loop(0, _NCHC, body, 0)
        plsc.subcore_barrier()

        pltpu.sync_copy(acc128.at[pl.ds(s * rows, rows)],
                        out1_hbm.at[pl.ds(c * _VP + s * rows, rows)])
        pltpu.sync_copy(acc8.at[pl.ds(s * rows, rows)],
                        out2_hbm.at[pl.ds(c * _VP + s * rows, rows)])

    return k(*mws, *cds, col, z128, z8)


def _node_body(h_ref, x_ref, s0_ref, s1_ref, c0_ref, c1_ref,
               wn1a_ref, wn1b_ref, bn1_ref, wn2_ref, bn2_ref,
               g_ref, b_ref, hout_ref, xout_ref):
    hb = h_ref[...]
    msg = s0_ref[...] + s1_ref[...]
    u = _silu(jnp.dot(hb, wn1a_ref[...], preferred_element_type=jnp.float32)
              + jnp.dot(msg, wn1b_ref[...], preferred_element_type=jnp.float32)
              + bn1_ref[...])
    hupd = jnp.dot(u, wn2_ref[...], preferred_element_type=jnp.float32) + bn2_ref[...]
    hh = hb + hupd
    mu = jnp.mean(hh, axis=1, keepdims=True)
    var = jnp.mean((hh - mu) ** 2, axis=1, keepdims=True)
    hout_ref[...] = (hh - mu) * lax.rsqrt(var + 1e-5) * g_ref[...] + b_ref[...]
    xout_ref[...] = x_ref[...] + c0_ref[:, 0:3] + c1_ref[:, 0:3]


def _node_call(h, x, s0, s1, c0, c1, wn1a, wn1b, bn1, wn2, bn2, g_ln, b_ln):
    full = lambda shape: pl.BlockSpec(shape, lambda i: tuple(0 for _ in shape))
    return pl.pallas_call(
        _node_body,
        grid=(V // _VB,),
        in_specs=[
            pl.BlockSpec((_VB, D), lambda i: (i, 0)),
            pl.BlockSpec((_VB, 3), lambda i: (i, 0)),
            pl.BlockSpec((_VB, D), lambda i: (i, 0)),
            pl.BlockSpec((_VB, D), lambda i: (i, 0)),
            pl.BlockSpec((_VB, 8), lambda i: (i, 0)),
            pl.BlockSpec((_VB, 8), lambda i: (i, 0)),
            full((D, D)), full((D, D)), full((1, D)),
            full((D, D)), full((1, D)), full((1, D)), full((1, D)),
        ],
        out_specs=[
            pl.BlockSpec((_VB, D), lambda i: (i, 0)),
            pl.BlockSpec((_VB, 3), lambda i: (i, 0)),
        ],
        out_shape=[
            jax.ShapeDtypeStruct((V, D), jnp.float32),
            jax.ShapeDtypeStruct((V, 3), jnp.float32),
        ],
    )(h, x, s0, s1, c0, c1, wn1a, wn1b, bn1, wn2, bn2, g_ln, b_ln)


def kernel(h, x, edge_index, curvature, pd1, pd2, We1, be1, We2, be2,
           Wcb1, bcb1, Wcb2, Wc11, bc11, Wc12, Wc21, bc21, Wc22,
           Wn1, bn1, Wn2, bn2, g_ln, b_ln, Wa1, ba1, Wa2, ba2):
    row = edge_index[0]
    col = edge_index[1]

    w1a = We1[0:D]
    w1b = We1[D:2 * D]
    w1l = We1[2 * D].reshape(1, D)

    pad1 = jnp.zeros((V, 1), jnp.float32)
    P = jnp.concatenate([x, pad1, pd1, pad1, pd2, pad1, curvature], axis=1)

    A, B = _prep_call(h, w1a, w1b)

    wh = jnp.concatenate([Wcb1, Wc11, Wc21], axis=1)
    bh = jnp.concatenate([bcb1, bc11, bc21]).reshape(1, 256)
    w16 = jnp.zeros((16, 32), jnp.float32).at[12:16].set(Wa1[4:8])

    mws, cds = [], []
    for ci in range(_NP):
        ar, bc, prg, pcg = _sc_gather(A, B, P, row, col, ci * _EC)
        mw, cd8 = _edge_call(
            ar, bc, prg, pcg,
            w1l, be1.reshape(1, D), We2, be2.reshape(1, D), wh, bh,
            Wcb2.reshape(1, D), Wc12.reshape(1, 64), Wc22.reshape(1, 64),
            Wa1, w16, ba1.reshape(1, 32), Wa2.reshape(1, 32),
            ba2.reshape(1, 1))
        mws.append(mw)
        cds.append(cd8)

    z128 = jnp.zeros((_VP, D), jnp.float32)
    z8 = jnp.zeros((_VP, 8), jnp.float32)
    s_part, c_part = _sc_scatter(mws, cds, col, z128, z8)

    h_out, x_new = _node_call(
        h, x, s_part[:V], s_part[_VP:_VP + V],
        c_part[:V], c_part[_VP:_VP + V],
        Wn1[0:D], Wn1[D:2 * D], bn1.reshape(1, D), Wn2, bn2.reshape(1, D),
        g_ln.reshape(1, D), b_ln.reshape(1, D))
    return (h_out, x_new)


# speedup vs baseline: 1.1210x; 1.1199x over previous
import functools

import jax
import jax.numpy as jnp
from jax import lax
from jax.experimental import pallas as pl
from jax.experimental.pallas import tpu as pltpu
from jax.experimental.pallas import tpu_sc as plsc

V = 10000
E = 320000
D = 128

_VB = 2000
_EB = 3200
_NW = 32
_C = 80
_NP = 5
_EC = E // _NP
_EWC = _EC // _NW
_NCHC = _EWC // _C
_VP = 10240


def _sigmoid(v):
    return 0.5 * jnp.tanh(0.5 * v) + 0.5


def _silu(v):
    return v * _sigmoid(v)


def _pack_bf16_pair(v):
    u = jax.lax.bitcast_convert_type(v, jnp.uint32)
    r = u + jnp.uint32(0x7FFF) + ((u >> 16) & jnp.uint32(1))
    hi = r[:, 0:64] & jnp.uint32(0xFFFF0000)
    lo = r[:, 64:128] >> 16
    return jax.lax.bitcast_convert_type(hi | lo, jnp.float32)


def _unpack_bf16_pair(p):
    u = jax.lax.bitcast_convert_type(p, jnp.uint32)
    hi = jax.lax.bitcast_convert_type(u & jnp.uint32(0xFFFF0000), jnp.float32)
    lo = jax.lax.bitcast_convert_type(u << 16, jnp.float32)
    return jnp.concatenate([hi, lo], axis=1)


def _prep_body(h_ref, w1a_ref, w1b_ref, a_ref, b_ref):
    hb = h_ref[...]
    a_ref[...] = _pack_bf16_pair(
        jnp.dot(hb, w1a_ref[...], preferred_element_type=jnp.float32))
    b_ref[...] = _pack_bf16_pair(
        jnp.dot(hb, w1b_ref[...], preferred_element_type=jnp.float32))


def _prep_call(h, w1a, w1b):
    return pl.pallas_call(
        _prep_body,
        grid=(V // _VB,),
        in_specs=[
            pl.BlockSpec((_VB, D), lambda i: (i, 0)),
            pl.BlockSpec((D, D), lambda i: (0, 0)),
            pl.BlockSpec((D, D), lambda i: (0, 0)),
        ],
        out_specs=[
            pl.BlockSpec((_VB, 64), lambda i: (i, 0)),
            pl.BlockSpec((_VB, 64), lambda i: (i, 0)),
        ],
        out_shape=[
            jax.ShapeDtypeStruct((V, 64), jnp.float32),
            jax.ShapeDtypeStruct((V, 64), jnp.float32),
        ],
    )(h, w1a, w1b)


def _sc_gather(a_tab, b_tab, p_tab, row, col, e0):
    mesh = plsc.VectorSubcoreMesh(core_axis_name="c", subcore_axis_name="s")

    @functools.partial(
        pl.kernel,
        mesh=mesh,
        compiler_params=pltpu.CompilerParams(use_tc_tiling_on_sc=False),
        out_type=[
            jax.ShapeDtypeStruct((_EC, 64), jnp.float32),
            jax.ShapeDtypeStruct((_EC, 64), jnp.float32),
            jax.ShapeDtypeStruct((_EC, 16), jnp.float32),
            jax.ShapeDtypeStruct((_EC, 16), jnp.float32),
        ],
        scratch_types=[
            pltpu.VMEM((_C,), jnp.int32),
            pltpu.VMEM((_C,), jnp.int32),
            pltpu.VMEM((_C, 64), jnp.float32),
            pltpu.VMEM((_C, 64), jnp.float32),
            pltpu.VMEM((_C, 16), jnp.float32),
            pltpu.VMEM((_C, 16), jnp.float32),
            pltpu.SemaphoreType.DMA,
            pltpu.SemaphoreType.DMA,
            pltpu.SemaphoreType.DMA,
            pltpu.SemaphoreType.DMA,
        ],
    )
    def k(a_hbm, b_hbm, p_hbm, row_hbm, col_hbm,
          ar_hbm, bc_hbm, pr_hbm, pc_hbm,
          idxr, idxc, bufa, bufb, bufpr, bufpc, sema, semb, semc, semd):
        w = lax.axis_index("s") * 2 + lax.axis_index("c")

        def body(i, carry):
            loc = w * _EWC + i * _C
            gbase = e0 + loc
            pltpu.sync_copy(row_hbm.at[pl.ds(gbase, _C)], idxr)
            pltpu.sync_copy(col_hbm.at[pl.ds(gbase, _C)], idxc)
            ca = pltpu.async_copy(a_hbm.at[idxr], bufa, sema)
            cb = pltpu.async_copy(b_hbm.at[idxc], bufb, semb)
            cpr = pltpu.async_copy(p_hbm.at[idxr], bufpr, semc)
            cpc = pltpu.async_copy(p_hbm.at[idxc], bufpc, semd)
            ca.wait()
            cb.wait()
            cpr.wait()
            cpc.wait()
            pltpu.sync_copy(bufa, ar_hbm.at[pl.ds(loc, _C)])
            pltpu.sync_copy(bufb, bc_hbm.at[pl.ds(loc, _C)])
            pltpu.sync_copy(bufpr, pr_hbm.at[pl.ds(loc, _C)])
            pltpu.sync_copy(bufpc, pc_hbm.at[pl.ds(loc, _C)])
            return carry

        lax.fori_loop(0, _NCHC, body, 0)

    return k(a_tab, b_tab, p_tab, row, col)


def _edge_body(ar_ref, bc_ref, pr_ref, pc_ref,
               w1l_ref, be1_ref, we2_ref, be2_ref,
               wh_ref, bh_ref, wcb2_ref, wc12_ref, wc22_ref,
               wa1_ref, w16_ref, ba1_ref, wa2_ref, ba2_ref,
               mw_ref, cd_ref):
    pr = pr_ref[...]
    pc = pc_ref[...]
    xr = pr[:, 0:3]
    pd1r = pr[:, 4:7]
    pd2r = pr[:, 8:11]
    curv = pr[:, 12:16]
    xc = pc[:, 0:3]
    diff = xc - xr
    d2 = jnp.sum(diff * diff, axis=1, keepdims=True)
    dist = jnp.maximum(jnp.sqrt(d2), 1e-6)
    inv_dist = 1.0 / dist
    e = diff * inv_dist
    c1 = jnp.sum(e * pd1r, axis=1, keepdims=True)
    c2 = jnp.sum(e * pd2r, axis=1, keepdims=True)

    msk = jnp.uint32(0xFFFF0000)
    a_u = jax.lax.bitcast_convert_type(ar_ref[...], jnp.uint32)
    b_u = jax.lax.bitcast_convert_type(bc_ref[...], jnp.uint32)
    f32 = lambda q: jax.lax.bitcast_convert_type(q, jnp.float32)
    t1h = (f32(a_u & msk) + f32(b_u & msk)
           + d2 * w1l_ref[:, 0:64] + be1_ref[:, 0:64])
    t1l = (f32(a_u << 16) + f32(b_u << 16)
           + d2 * w1l_ref[:, 64:128] + be1_ref[:, 64:128])
    uh = _silu(t1h)
    ul = _silu(t1l)
    m_ij = _silu(jnp.dot(uh, we2_ref[0:64], preferred_element_type=jnp.float32)
                 + jnp.dot(ul, we2_ref[64:128], preferred_element_type=jnp.float32)
                 + be2_ref[...])

    k1 = curv[:, 0:1]
    k2 = curv[:, 1:2]
    hid = (jnp.dot(pr, w16_ref[...], preferred_element_type=jnp.float32)
           + ba1_ref[...])
    terms = (jnp.abs(c1), jnp.abs(c2), c1 * k1, c2 * k2, dist)
    for kk, t in zip((0, 1, 2, 3, 8), terms):
        hid = hid + t * wa1_ref[kk:kk + 1, :]
    hid = _silu(hid)
    w_an = _sigmoid(jnp.sum(hid * wa2_ref[...], axis=1, keepdims=True)
                    + ba2_ref[...]) + 0.5

    m_w = m_ij * w_an

    ch = _silu(jnp.dot(m_w, wh_ref[...], preferred_element_type=jnp.float32)
               + bh_ref[...])
    alpha_b = jnp.tanh(jnp.sum(ch[:, 0:128] * wcb2_ref[...],
                               axis=1, keepdims=True))
    alpha1 = jnp.tanh(jnp.sum(ch[:, 128:192] * wc12_ref[...],
                              axis=1, keepdims=True))
    alpha2 = jnp.tanh(jnp.sum(ch[:, 192:256] * wc22_ref[...],
                              axis=1, keepdims=True))

    inv_dn = 1.0 / (dist + 1.0)
    cd = ((alpha_b * inv_dn) * diff + (alpha1 * c1) * pd1r
          + (alpha2 * c2) * pd2r)

    mw_ref[...] = m_w
    cd_ref[...] = jnp.concatenate(
        [cd, jnp.zeros((cd.shape[0], 5), jnp.float32)], axis=1)


def _edge_call(ar, bc, prg, pcg, w1l, be1, we2, be2, wh, bh,
               wcb2r, wc12r, wc22r, wa1, w16, ba1, wa2r, ba2):
    full = lambda shape: pl.BlockSpec(shape, lambda i: tuple(0 for _ in shape))
    return pl.pallas_call(
        _edge_body,
        grid=(_EC // _EB,),
        in_specs=[
            pl.BlockSpec((_EB, 64), lambda i: (i, 0)),
            pl.BlockSpec((_EB, 64), lambda i: (i, 0)),
            pl.BlockSpec((_EB, 16), lambda i: (i, 0)),
            pl.BlockSpec((_EB, 16), lambda i: (i, 0)),
            full((1, D)), full((1, D)), full((D, D)), full((1, D)),
            full((D, 256)), full((1, 256)),
            full((1, D)), full((1, 64)), full((1, 64)),
            full((9, 32)), full((16, 32)), full((1, 32)), full((1, 32)),
            full((1, 1)),
        ],
        out_specs=[
            pl.BlockSpec((_EB, D), lambda i: (i, 0)),
            pl.BlockSpec((_EB, 8), lambda i: (i, 0)),
        ],
        out_shape=[
            jax.ShapeDtypeStruct((_EC, D), jnp.float32),
            jax.ShapeDtypeStruct((_EC, 8), jnp.float32),
        ],
    )(ar, bc, prg, pcg, w1l, be1, we2, be2, wh, bh,
      wcb2r, wc12r, wc22r, wa1, w16, ba1, wa2r, ba2)


def _sc_scatter(mws, cds, col, z128, z8):
    mesh = plsc.VectorSubcoreMesh(core_axis_name="c", subcore_axis_name="s")
    rows = _VP // 16

    @functools.partial(
        pl.kernel,
        mesh=mesh,
        compiler_params=pltpu.CompilerParams(use_tc_tiling_on_sc=False),
        out_type=[
            jax.ShapeDtypeStruct((2 * _VP, D), jnp.float32),
            jax.ShapeDtypeStruct((2 * _VP, 8), jnp.float32),
        ],
        scratch_types=[
            pltpu.VMEM((_C,), jnp.int32),
            pltpu.VMEM((_C, D), jnp.float32),
            pltpu.VMEM((_C, 8), jnp.float32),
            pltpu.VMEM_SHARED((_VP, D), jnp.float32),
            pltpu.VMEM_SHARED((_VP, 8), jnp.float32),
            pltpu.SemaphoreType.DMA,
            pltpu.SemaphoreType.DMA,
        ],
    )
    def k(mw0, mw1, mw2, mw3, mw4, cd0, cd1, cd2, cd3, cd4,
          col_hbm, z128_hbm, z8_hbm, out1_hbm, out2_hbm,
          idxc, bufm, bufc, acc128, acc8, sem1, sem2):
        mw_hbms = (mw0, mw1, mw2, mw3, mw4)
        cd_hbms = (cd0, cd1, cd2, cd3, cd4)
        c = lax.axis_index("c")
        s = lax.axis_index("s")
        w = s * 2 + c

        pltpu.sync_copy(z128_hbm.at[pl.ds(s * rows, rows)],
                        acc128.at[pl.ds(s * rows, rows)])
        pltpu.sync_copy(z8_hbm.at[pl.ds(s * rows, rows)],
                        acc8.at[pl.ds(s * rows, rows)])
        plsc.subcore_barrier()

        for ci in range(_NP):
            mw_hbm = mw_hbms[ci]
            cd_hbm = cd_hbms[ci]

            def body(i, carry):
                loc = w * _EWC + i * _C
                pltpu.sync_copy(col_hbm.at[pl.ds(ci * _EC + loc, _C)], idxc)
                cm = pltpu.async_copy(mw_hbm.at[pl.ds(loc, _C)], bufm, sem1)
                cc = pltpu.async_copy(cd_hbm.at[pl.ds(loc, _C)], bufc, sem2)
                cm.wait()
                cc.wait()
                pltpu.sync_copy(bufm, acc128.at[idxc], add=True)
                pltpu.sync_copy(bufc, acc8.at[idxc], add=True)
                return carry

            lax.fori_loop(0, _NCHC, body, 0)
        plsc.subcore_barrier()

        pltpu.sync_copy(acc128.at[pl.ds(s * rows, rows)],
                        out1_hbm.at[pl.ds(c * _VP + s * rows, rows)])
        pltpu.sync_copy(acc8.at[pl.ds(s * rows, rows)],
                        out2_hbm.at[pl.ds(c * _VP + s * rows, rows)])

    return k(*mws, *cds, col, z128, z8)


def _node_body(h_ref, x_ref, s0_ref, s1_ref, c0_ref, c1_ref,
               wn1a_ref, wn1b_ref, bn1_ref, wn2_ref, bn2_ref,
               g_ref, b_ref, hout_ref, xout_ref):
    hb = h_ref[...]
    msg = s0_ref[...] + s1_ref[...]
    u = _silu(jnp.dot(hb, wn1a_ref[...], preferred_element_type=jnp.float32)
              + jnp.dot(msg, wn1b_ref[...], preferred_element_type=jnp.float32)
              + bn1_ref[...])
    hupd = jnp.dot(u, wn2_ref[...], preferred_element_type=jnp.float32) + bn2_ref[...]
    hh = hb + hupd
    mu = jnp.mean(hh, axis=1, keepdims=True)
    var = jnp.mean((hh - mu) ** 2, axis=1, keepdims=True)
    hout_ref[...] = (hh - mu) * lax.rsqrt(var + 1e-5) * g_ref[...] + b_ref[...]
    xout_ref[...] = x_ref[...] + c0_ref[:, 0:3] + c1_ref[:, 0:3]


def _node_call(h, x, s0, s1, c0, c1, wn1a, wn1b, bn1, wn2, bn2, g_ln, b_ln):
    full = lambda shape: pl.BlockSpec(shape, lambda i: tuple(0 for _ in shape))
    return pl.pallas_call(
        _node_body,
        grid=(V // _VB,),
        in_specs=[
            pl.BlockSpec((_VB, D), lambda i: (i, 0)),
            pl.BlockSpec((_VB, 3), lambda i: (i, 0)),
            pl.BlockSpec((_VB, D), lambda i: (i, 0)),
            pl.BlockSpec((_VB, D), lambda i: (i, 0)),
            pl.BlockSpec((_VB, 8), lambda i: (i, 0)),
            pl.BlockSpec((_VB, 8), lambda i: (i, 0)),
            full((D, D)), full((D, D)), full((1, D)),
            full((D, D)), full((1, D)), full((1, D)), full((1, D)),
        ],
        out_specs=[
            pl.BlockSpec((_VB, D), lambda i: (i, 0)),
            pl.BlockSpec((_VB, 3), lambda i: (i, 0)),
        ],
        out_shape=[
            jax.ShapeDtypeStruct((V, D), jnp.float32),
            jax.ShapeDtypeStruct((V, 3), jnp.float32),
        ],
    )(h, x, s0, s1, c0, c1, wn1a, wn1b, bn1, wn2, bn2, g_ln, b_ln)


def kernel(h, x, edge_index, curvature, pd1, pd2, We1, be1, We2, be2,
           Wcb1, bcb1, Wcb2, Wc11, bc11, Wc12, Wc21, bc21, Wc22,
           Wn1, bn1, Wn2, bn2, g_ln, b_ln, Wa1, ba1, Wa2, ba2):
    row = edge_index[0]
    col = edge_index[1]

    w1a = We1[0:D]
    w1b = We1[D:2 * D]
    w1l = We1[2 * D].reshape(1, D)

    pad1 = jnp.zeros((V, 1), jnp.float32)
    P = jnp.concatenate([x, pad1, pd1, pad1, pd2, pad1, curvature], axis=1)

    A, B = _prep_call(h, w1a, w1b)

    wh = jnp.concatenate([Wcb1, Wc11, Wc21], axis=1)
    bh = jnp.concatenate([bcb1, bc11, bc21]).reshape(1, 256)
    w16 = jnp.zeros((16, 32), jnp.float32).at[12:16].set(Wa1[4:8])

    mws, cds = [], []
    for ci in range(_NP):
        ar, bc, prg, pcg = _sc_gather(A, B, P, row, col, ci * _EC)
        mw, cd8 = _edge_call(
            ar, bc, prg, pcg,
            w1l, be1.reshape(1, D), We2, be2.reshape(1, D), wh, bh,
            Wcb2.reshape(1, D), Wc12.reshape(1, 64), Wc22.reshape(1, 64),
            Wa1, w16, ba1.reshape(1, 32), Wa2.reshape(1, 32),
            ba2.reshape(1, 1))
        mws.append(mw)
        cds.append(cd8)

    z128 = jnp.zeros((_VP, D), jnp.float32)
    z8 = jnp.zeros((_VP, 8), jnp.float32)
    s_part, c_part = _sc_scatter(mws, cds, col, z128, z8)

    h_out, x_new = _node_call(
        h, x, s_part[:V], s_part[_VP:_VP + V],
        c_part[:V], c_part[_VP:_VP + V],
        Wn1[0:D], Wn1[D:2 * D], bn1.reshape(1, D), Wn2, bn2.reshape(1, D),
        g_ln.reshape(1, D), b_ln.reshape(1, D))
    return (h_out, x_new)


# speedup vs baseline: 1.1620x; 1.0365x over previous
import functools

import jax
import jax.numpy as jnp
from jax import lax
from jax.experimental import pallas as pl
from jax.experimental.pallas import tpu as pltpu
from jax.experimental.pallas import tpu_sc as plsc

V = 10000
E = 320000
D = 128

_VB = 2000
_EB = 3200
_NW = 32
_C = 80
_NP = 5
_EC = E // _NP
_EWC = _EC // _NW
_NCHC = _EWC // _C
_VP = 10240


def _sigmoid(v):
    return 0.5 * jnp.tanh(0.5 * v) + 0.5


def _silu(v):
    return v * _sigmoid(v)


def _pack_bf16_pair(v):
    u = jax.lax.bitcast_convert_type(v, jnp.uint32)
    r = u + jnp.uint32(0x7FFF) + ((u >> 16) & jnp.uint32(1))
    hi = r[:, 0:64] & jnp.uint32(0xFFFF0000)
    lo = r[:, 64:128] >> 16
    return jax.lax.bitcast_convert_type(hi | lo, jnp.float32)


def _unpack_bf16_pair(p):
    u = jax.lax.bitcast_convert_type(p, jnp.uint32)
    hi = jax.lax.bitcast_convert_type(u & jnp.uint32(0xFFFF0000), jnp.float32)
    lo = jax.lax.bitcast_convert_type(u << 16, jnp.float32)
    return jnp.concatenate([hi, lo], axis=1)


def _prep_body(h_ref, w1a_ref, w1b_ref, a_ref, b_ref):
    hb = h_ref[...]
    a_ref[...] = _pack_bf16_pair(
        jnp.dot(hb, w1a_ref[...], preferred_element_type=jnp.float32))
    b_ref[...] = _pack_bf16_pair(
        jnp.dot(hb, w1b_ref[...], preferred_element_type=jnp.float32))


def _prep_call(h, w1a, w1b):
    return pl.pallas_call(
        _prep_body,
        grid=(V // _VB,),
        in_specs=[
            pl.BlockSpec((_VB, D), lambda i: (i, 0)),
            pl.BlockSpec((D, D), lambda i: (0, 0)),
            pl.BlockSpec((D, D), lambda i: (0, 0)),
        ],
        out_specs=[
            pl.BlockSpec((_VB, 64), lambda i: (i, 0)),
            pl.BlockSpec((_VB, 64), lambda i: (i, 0)),
        ],
        out_shape=[
            jax.ShapeDtypeStruct((V, 64), jnp.float32),
            jax.ShapeDtypeStruct((V, 64), jnp.float32),
        ],
    )(h, w1a, w1b)


def _sc_gather(a_tab, b_tab, p_tab, row, col, e0):
    mesh = plsc.VectorSubcoreMesh(core_axis_name="c", subcore_axis_name="s")

    @functools.partial(
        pl.kernel,
        mesh=mesh,
        compiler_params=pltpu.CompilerParams(use_tc_tiling_on_sc=False),
        out_type=[
            jax.ShapeDtypeStruct((_EC, 64), jnp.float32),
            jax.ShapeDtypeStruct((_EC, 64), jnp.float32),
            jax.ShapeDtypeStruct((_EC, 16), jnp.float32),
            jax.ShapeDtypeStruct((_EC, 16), jnp.float32),
        ],
        scratch_types=[
            pltpu.VMEM((_C,), jnp.int32),
            pltpu.VMEM((_C,), jnp.int32),
            pltpu.VMEM((_C, 64), jnp.float32),
            pltpu.VMEM((_C, 64), jnp.float32),
            pltpu.VMEM((_C, 16), jnp.float32),
            pltpu.VMEM((_C, 16), jnp.float32),
            pltpu.SemaphoreType.DMA,
            pltpu.SemaphoreType.DMA,
            pltpu.SemaphoreType.DMA,
            pltpu.SemaphoreType.DMA,
        ],
    )
    def k(a_hbm, b_hbm, p_hbm, row_hbm, col_hbm,
          ar_hbm, bc_hbm, pr_hbm, pc_hbm,
          idxr, idxc, bufa, bufb, bufpr, bufpc, sema, semb, semc, semd):
        w = lax.axis_index("s") * 2 + lax.axis_index("c")

        def body(i, carry):
            loc = w * _EWC + i * _C
            gbase = e0 + loc
            pltpu.sync_copy(row_hbm.at[pl.ds(gbase, _C)], idxr)
            pltpu.sync_copy(col_hbm.at[pl.ds(gbase, _C)], idxc)
            ca = pltpu.async_copy(a_hbm.at[idxr], bufa, sema)
            cb = pltpu.async_copy(b_hbm.at[idxc], bufb, semb)
            cpr = pltpu.async_copy(p_hbm.at[idxr], bufpr, semc)
            cpc = pltpu.async_copy(p_hbm.at[idxc], bufpc, semd)
            ca.wait()
            cb.wait()
            cpr.wait()
            cpc.wait()
            pltpu.sync_copy(bufa, ar_hbm.at[pl.ds(loc, _C)])
            pltpu.sync_copy(bufb, bc_hbm.at[pl.ds(loc, _C)])
            pltpu.sync_copy(bufpr, pr_hbm.at[pl.ds(loc, _C)])
            pltpu.sync_copy(bufpc, pc_hbm.at[pl.ds(loc, _C)])
            return carry

        lax.fori_loop(0, _NCHC, body, 0)

    return k(a_tab, b_tab, p_tab, row, col)


def _edge_body(ar_ref, bc_ref, pr_ref, pc_ref,
               w1l_ref, be1_ref, we2_ref, be2_ref,
               wh_ref, bh_ref, wsel_ref,
               wa1_ref, w16_ref, ba1_ref, wa2_ref, ba2_ref,
               mw_ref, cd_ref):
    pr = pr_ref[...]
    pc = pc_ref[...]
    xr = pr[:, 0:3]
    pd1r = pr[:, 4:7]
    pd2r = pr[:, 8:11]
    curv = pr[:, 12:16]
    xc = pc[:, 0:3]
    diff = xc - xr
    d2 = jnp.sum(diff * diff, axis=1, keepdims=True)
    d2c = jnp.maximum(d2, 1e-12)
    inv_dist = lax.rsqrt(d2c)
    dist = d2c * inv_dist
    e = diff * inv_dist
    c1 = jnp.sum(e * pd1r, axis=1, keepdims=True)
    c2 = jnp.sum(e * pd2r, axis=1, keepdims=True)

    msk = jnp.uint32(0xFFFF0000)
    a_u = jax.lax.bitcast_convert_type(ar_ref[...], jnp.uint32)
    b_u = jax.lax.bitcast_convert_type(bc_ref[...], jnp.uint32)
    f32 = lambda q: jax.lax.bitcast_convert_type(q, jnp.float32)
    t1h = (f32(a_u & msk) + f32(b_u & msk)
           + d2 * w1l_ref[:, 0:64] + be1_ref[:, 0:64])
    t1l = (f32(a_u << 16) + f32(b_u << 16)
           + d2 * w1l_ref[:, 64:128] + be1_ref[:, 64:128])
    uh = _silu(t1h)
    ul = _silu(t1l)
    m_ij = _silu(jnp.dot(uh, we2_ref[0:64], preferred_element_type=jnp.float32)
                 + jnp.dot(ul, we2_ref[64:128], preferred_element_type=jnp.float32)
                 + be2_ref[...])

    k1 = curv[:, 0:1]
    k2 = curv[:, 1:2]
    hid = (jnp.dot(pr, w16_ref[...], preferred_element_type=jnp.float32)
           + ba1_ref[...])
    terms = (jnp.abs(c1), jnp.abs(c2), c1 * k1, c2 * k2, dist)
    for kk, t in zip((0, 1, 2, 3, 8), terms):
        hid = hid + t * wa1_ref[kk:kk + 1, :]
    hid = _silu(hid)
    w_an = _sigmoid(jnp.sum(hid * wa2_ref[...], axis=1, keepdims=True)
                    + ba2_ref[...]) + 0.5

    m_w = m_ij * w_an

    ch = _silu(jnp.dot(m_w, wh_ref[...], preferred_element_type=jnp.float32)
               + bh_ref[...])
    al = jnp.tanh(jnp.dot(ch, wsel_ref[...],
                          preferred_element_type=jnp.float32)[:, 0:3])
    alpha_b = al[:, 0:1]
    alpha1 = al[:, 1:2]
    alpha2 = al[:, 2:3]

    inv_dn = 1.0 / (dist + 1.0)
    cd = ((alpha_b * inv_dn) * diff + (alpha1 * c1) * pd1r
          + (alpha2 * c2) * pd2r)

    mw_ref[...] = m_w
    cd_ref[...] = jnp.concatenate(
        [cd, jnp.zeros((cd.shape[0], 5), jnp.float32)], axis=1)


def _edge_call(ar, bc, prg, pcg, w1l, be1, we2, be2, wh, bh,
               wsel, wa1, w16, ba1, wa2r, ba2):
    full = lambda shape: pl.BlockSpec(shape, lambda i: tuple(0 for _ in shape))
    return pl.pallas_call(
        _edge_body,
        grid=(_EC // _EB,),
        in_specs=[
            pl.BlockSpec((_EB, 64), lambda i: (i, 0)),
            pl.BlockSpec((_EB, 64), lambda i: (i, 0)),
            pl.BlockSpec((_EB, 16), lambda i: (i, 0)),
            pl.BlockSpec((_EB, 16), lambda i: (i, 0)),
            full((1, D)), full((1, D)), full((D, D)), full((1, D)),
            full((D, 256)), full((1, 256)),
            full((256, D)),
            full((9, 32)), full((16, 32)), full((1, 32)), full((1, 32)),
            full((1, 1)),
        ],
        out_specs=[
            pl.BlockSpec((_EB, D), lambda i: (i, 0)),
            pl.BlockSpec((_EB, 8), lambda i: (i, 0)),
        ],
        out_shape=[
            jax.ShapeDtypeStruct((_EC, D), jnp.float32),
            jax.ShapeDtypeStruct((_EC, 8), jnp.float32),
        ],
    )(ar, bc, prg, pcg, w1l, be1, we2, be2, wh, bh,
      wsel, wa1, w16, ba1, wa2r, ba2)


def _sc_scatter(mws, cds, col, z128, z8):
    mesh = plsc.VectorSubcoreMesh(core_axis_name="c", subcore_axis_name="s")
    rows = _VP // 16

    @functools.partial(
        pl.kernel,
        mesh=mesh,
        compiler_params=pltpu.CompilerParams(use_tc_tiling_on_sc=False),
        out_type=[
            jax.ShapeDtypeStruct((2 * _VP, D), jnp.float32),
            jax.ShapeDtypeStruct((2 * _VP, 8), jnp.float32),
        ],
        scratch_types=[
            pltpu.VMEM((_C,), jnp.int32),
            pltpu.VMEM((_C, D), jnp.float32),
            pltpu.VMEM((_C, 8), jnp.float32),
            pltpu.VMEM_SHARED((_VP, D), jnp.float32),
            pltpu.VMEM_SHARED((_VP, 8), jnp.float32),
            pltpu.SemaphoreType.DMA,
            pltpu.SemaphoreType.DMA,
        ],
    )
    def k(mw0, mw1, mw2, mw3, mw4, cd0, cd1, cd2, cd3, cd4,
          col_hbm, z128_hbm, z8_hbm, out1_hbm, out2_hbm,
          idxc, bufm, bufc, acc128, acc8, sem1, sem2):
        mw_hbms = (mw0, mw1, mw2, mw3, mw4)
        cd_hbms = (cd0, cd1, cd2, cd3, cd4)
        c = lax.axis_index("c")
        s = lax.axis_index("s")
        w = s * 2 + c

        pltpu.sync_copy(z128_hbm.at[pl.ds(s * rows, rows)],
                        acc128.at[pl.ds(s * rows, rows)])
        pltpu.sync_copy(z8_hbm.at[pl.ds(s * rows, rows)],
                        acc8.at[pl.ds(s * rows, rows)])
        plsc.subcore_barrier()

        for ci in range(_NP):
            mw_hbm = mw_hbms[ci]
            cd_hbm = cd_hbms[ci]

            def body(i, carry):
                loc = w * _EWC + i * _C
                pltpu.sync_copy(col_hbm.at[pl.ds(ci * _EC + loc, _C)], idxc)
                cm = pltpu.async_copy(mw_hbm.at[pl.ds(loc, _C)], bufm, sem1)
                cc = pltpu.async_copy(cd_hbm.at[pl.ds(loc, _C)], bufc, sem2)
                cm.wait()
                cc.wait()
                pltpu.sync_copy(bufm, acc128.at[idxc], add=True)
                pltpu.sync_copy(bufc, acc8.at[idxc], add=True)
                return carry

            lax.fori_loop(0, _NCHC, body, 0)
        plsc.subcore_barrier()

        pltpu.sync_copy(acc128.at[pl.ds(s * rows, rows)],
                        out1_hbm.at[pl.ds(c * _VP + s * rows, rows)])
        pltpu.sync_copy(acc8.at[pl.ds(s * rows, rows)],
                        out2_hbm.at[pl.ds(c * _VP + s * rows, rows)])

    return k(*mws, *cds, col, z128, z8)


def _node_body(h_ref, x_ref, s0_ref, s1_ref, c0_ref, c1_ref,
               wn1a_ref, wn1b_ref, bn1_ref, wn2_ref, bn2_ref,
               g_ref, b_ref, hout_ref, xout_ref):
    hb = h_ref[...]
    msg = s0_ref[...] + s1_ref[...]
    u = _silu(jnp.dot(hb, wn1a_ref[...], preferred_element_type=jnp.float32)
              + jnp.dot(msg, wn1b_ref[...], preferred_element_type=jnp.float32)
              + bn1_ref[...])
    hupd = jnp.dot(u, wn2_ref[...], preferred_element_type=jnp.float32) + bn2_ref[...]
    hh = hb + hupd
    mu = jnp.mean(hh, axis=1, keepdims=True)
    var = jnp.mean((hh - mu) ** 2, axis=1, keepdims=True)
    hout_ref[...] = (hh - mu) * lax.rsqrt(var + 1e-5) * g_ref[...] + b_ref[...]
    xout_ref[...] = x_ref[...] + c0_ref[:, 0:3] + c1_ref[:, 0:3]


def _node_call(h, x, s0, s1, c0, c1, wn1a, wn1b, bn1, wn2, bn2, g_ln, b_ln):
    full = lambda shape: pl.BlockSpec(shape, lambda i: tuple(0 for _ in shape))
    return pl.pallas_call(
        _node_body,
        grid=(V // _VB,),
        in_specs=[
            pl.BlockSpec((_VB, D), lambda i: (i, 0)),
            pl.BlockSpec((_VB, 3), lambda i: (i, 0)),
            pl.BlockSpec((_VB, D), lambda i: (i, 0)),
            pl.BlockSpec((_VB, D), lambda i: (i, 0)),
            pl.BlockSpec((_VB, 8), lambda i: (i, 0)),
            pl.BlockSpec((_VB, 8), lambda i: (i, 0)),
            full((D, D)), full((D, D)), full((1, D)),
            full((D, D)), full((1, D)), full((1, D)), full((1, D)),
        ],
        out_specs=[
            pl.BlockSpec((_VB, D), lambda i: (i, 0)),
            pl.BlockSpec((_VB, 3), lambda i: (i, 0)),
        ],
        out_shape=[
            jax.ShapeDtypeStruct((V, D), jnp.float32),
            jax.ShapeDtypeStruct((V, 3), jnp.float32),
        ],
    )(h, x, s0, s1, c0, c1, wn1a, wn1b, bn1, wn2, bn2, g_ln, b_ln)


def kernel(h, x, edge_index, curvature, pd1, pd2, We1, be1, We2, be2,
           Wcb1, bcb1, Wcb2, Wc11, bc11, Wc12, Wc21, bc21, Wc22,
           Wn1, bn1, Wn2, bn2, g_ln, b_ln, Wa1, ba1, Wa2, ba2):
    row = edge_index[0]
    col = edge_index[1]

    w1a = We1[0:D]
    w1b = We1[D:2 * D]
    w1l = We1[2 * D].reshape(1, D)

    pad1 = jnp.zeros((V, 1), jnp.float32)
    P = jnp.concatenate([x, pad1, pd1, pad1, pd2, pad1, curvature], axis=1)

    A, B = _prep_call(h, w1a, w1b)

    wh = jnp.concatenate([Wcb1, Wc11, Wc21], axis=1)
    bh = jnp.concatenate([bcb1, bc11, bc21]).reshape(1, 256)
    w16 = jnp.zeros((16, 32), jnp.float32).at[12:16].set(Wa1[4:8])
    wsel = (jnp.zeros((256, D), jnp.float32)
            .at[0:128, 0].set(Wcb2[:, 0])
            .at[128:192, 1].set(Wc12[:, 0])
            .at[192:256, 2].set(Wc22[:, 0]))

    mws, cds = [], []
    for ci in range(_NP):
        ar, bc, prg, pcg = _sc_gather(A, B, P, row, col, ci * _EC)
        mw, cd8 = _edge_call(
            ar, bc, prg, pcg,
            w1l, be1.reshape(1, D), We2, be2.reshape(1, D), wh, bh,
            wsel, Wa1, w16, ba1.reshape(1, 32), Wa2.reshape(1, 32),
            ba2.reshape(1, 1))
        mws.append(mw)
        cds.append(cd8)

    z128 = jnp.zeros((_VP, D), jnp.float32)
    z8 = jnp.zeros((_VP, 8), jnp.float32)
    s_part, c_part = _sc_scatter(mws, cds, col, z128, z8)

    h_out, x_new = _node_call(
        h, x, s_part[:V], s_part[_VP:_VP + V],
        c_part[:V], c_part[_VP:_VP + V],
        Wn1[0:D], Wn1[D:2 * D], bn1.reshape(1, D), Wn2, bn2.reshape(1, D),
        g_ln.reshape(1, D), b_ln.reshape(1, D))
    return (h_out, x_new)


# speedup vs baseline: 1.1727x; 1.0092x over previous
import functools

import jax
import jax.numpy as jnp
from jax import lax
from jax.experimental import pallas as pl
from jax.experimental.pallas import tpu as pltpu
from jax.experimental.pallas import tpu_sc as plsc

V = 10000
E = 320000
D = 128

_VB = 2000
_EB = 3200
_NW = 32
_C = 80
_NP = 5
_EC = E // _NP
_EWC = _EC // _NW
_NCHC = _EWC // _C
_VP = 10240


def _sigmoid(v):
    return 0.5 * jnp.tanh(0.5 * v) + 0.5


def _silu(v):
    return v * _sigmoid(v)


def _pack_bf16_pair(v):
    u = jax.lax.bitcast_convert_type(v, jnp.uint32)
    r = u + jnp.uint32(0x7FFF) + ((u >> 16) & jnp.uint32(1))
    hi = r[:, 0:64] & jnp.uint32(0xFFFF0000)
    lo = r[:, 64:128] >> 16
    return jax.lax.bitcast_convert_type(hi | lo, jnp.float32)


def _unpack_bf16_pair(p):
    u = jax.lax.bitcast_convert_type(p, jnp.uint32)
    hi = jax.lax.bitcast_convert_type(u & jnp.uint32(0xFFFF0000), jnp.float32)
    lo = jax.lax.bitcast_convert_type(u << 16, jnp.float32)
    return jnp.concatenate([hi, lo], axis=1)


def _prep_body(h_ref, w1a_ref, w1b_ref, a_ref, b_ref):
    hb = h_ref[...]
    a_ref[...] = _pack_bf16_pair(
        jnp.dot(hb, w1a_ref[...], preferred_element_type=jnp.float32))
    b_ref[...] = _pack_bf16_pair(
        jnp.dot(hb, w1b_ref[...], preferred_element_type=jnp.float32))


def _prep_call(h, w1a, w1b):
    return pl.pallas_call(
        _prep_body,
        grid=(V // _VB,),
        in_specs=[
            pl.BlockSpec((_VB, D), lambda i: (i, 0)),
            pl.BlockSpec((D, D), lambda i: (0, 0)),
            pl.BlockSpec((D, D), lambda i: (0, 0)),
        ],
        out_specs=[
            pl.BlockSpec((_VB, 64), lambda i: (i, 0)),
            pl.BlockSpec((_VB, 64), lambda i: (i, 0)),
        ],
        out_shape=[
            jax.ShapeDtypeStruct((V, 64), jnp.float32),
            jax.ShapeDtypeStruct((V, 64), jnp.float32),
        ],
    )(h, w1a, w1b)


def _sc_gather(a_tab, b_tab, p_tab, row, col, e0):
    mesh = plsc.VectorSubcoreMesh(core_axis_name="c", subcore_axis_name="s")

    @functools.partial(
        pl.kernel,
        mesh=mesh,
        compiler_params=pltpu.CompilerParams(use_tc_tiling_on_sc=False),
        out_type=[
            jax.ShapeDtypeStruct((_EC, 64), jnp.float32),
            jax.ShapeDtypeStruct((_EC, 64), jnp.float32),
            jax.ShapeDtypeStruct((_EC, 16), jnp.float32),
            jax.ShapeDtypeStruct((_EC, 16), jnp.float32),
        ],
        scratch_types=[
            [pltpu.VMEM((_C,), jnp.int32)] * 2,
            [pltpu.VMEM((_C,), jnp.int32)] * 2,
            [pltpu.VMEM((_C, 64), jnp.float32)] * 2,
            [pltpu.VMEM((_C, 64), jnp.float32)] * 2,
            [pltpu.VMEM((_C, 16), jnp.float32)] * 2,
            [pltpu.VMEM((_C, 16), jnp.float32)] * 2,
            [pltpu.SemaphoreType.DMA] * 2,
            [pltpu.SemaphoreType.DMA] * 2,
            [pltpu.SemaphoreType.DMA] * 2,
            [pltpu.SemaphoreType.DMA] * 2,
        ],
    )
    def k(a_hbm, b_hbm, p_hbm, row_hbm, col_hbm,
          ar_hbm, bc_hbm, pr_hbm, pc_hbm,
          idxr, idxc, bufa, bufb, bufpr, bufpc, sema, semb, semc, semd):
        w = lax.axis_index("s") * 2 + lax.axis_index("c")

        def load_idx(i, s):
            gbase = e0 + w * _EWC + i * _C
            pltpu.sync_copy(row_hbm.at[pl.ds(gbase, _C)], idxr[s])
            pltpu.sync_copy(col_hbm.at[pl.ds(gbase, _C)], idxc[s])

        def issue(s):
            pltpu.async_copy(a_hbm.at[idxr[s]], bufa[s], sema[s])
            pltpu.async_copy(b_hbm.at[idxc[s]], bufb[s], semb[s])
            pltpu.async_copy(p_hbm.at[idxr[s]], bufpr[s], semc[s])
            pltpu.async_copy(p_hbm.at[idxc[s]], bufpc[s], semd[s])

        def drain(s):
            pltpu.make_async_copy(a_hbm.at[idxr[s]], bufa[s], sema[s]).wait()
            pltpu.make_async_copy(b_hbm.at[idxc[s]], bufb[s], semb[s]).wait()
            pltpu.make_async_copy(p_hbm.at[idxr[s]], bufpr[s], semc[s]).wait()
            pltpu.make_async_copy(p_hbm.at[idxc[s]], bufpc[s], semd[s]).wait()

        def writeback(i, s):
            loc = w * _EWC + i * _C
            pltpu.sync_copy(bufa[s], ar_hbm.at[pl.ds(loc, _C)])
            pltpu.sync_copy(bufb[s], bc_hbm.at[pl.ds(loc, _C)])
            pltpu.sync_copy(bufpr[s], pr_hbm.at[pl.ds(loc, _C)])
            pltpu.sync_copy(bufpc[s], pc_hbm.at[pl.ds(loc, _C)])

        load_idx(0, 0)
        issue(0)

        def body(j, carry):
            c1 = 2 * j + 1
            c2 = 2 * j + 2
            load_idx(c1, 1)
            issue(1)
            drain(0)
            writeback(c1 - 1, 0)
            load_idx(c2, 0)
            issue(0)
            drain(1)
            writeback(c1, 1)
            return carry

        lax.fori_loop(0, (_NCHC - 1) // 2, body, 0)
        drain(0)
        writeback(_NCHC - 1, 0)

    return k(a_tab, b_tab, p_tab, row, col)


def _edge_body(ar_ref, bc_ref, pr_ref, pc_ref,
               w1l_ref, be1_ref, we2_ref, be2_ref,
               wh_ref, bh_ref, wsel_ref,
               wa1_ref, w16_ref, ba1_ref, wa2_ref, ba2_ref,
               mw_ref, cd_ref):
    pr = pr_ref[...]
    pc = pc_ref[...]
    xr = pr[:, 0:3]
    pd1r = pr[:, 4:7]
    pd2r = pr[:, 8:11]
    curv = pr[:, 12:16]
    xc = pc[:, 0:3]
    diff = xc - xr
    d2 = jnp.sum(diff * diff, axis=1, keepdims=True)
    d2c = jnp.maximum(d2, 1e-12)
    inv_dist = lax.rsqrt(d2c)
    dist = d2c * inv_dist
    e = diff * inv_dist
    c1 = jnp.sum(e * pd1r, axis=1, keepdims=True)
    c2 = jnp.sum(e * pd2r, axis=1, keepdims=True)

    msk = jnp.uint32(0xFFFF0000)
    a_u = jax.lax.bitcast_convert_type(ar_ref[...], jnp.uint32)
    b_u = jax.lax.bitcast_convert_type(bc_ref[...], jnp.uint32)
    f32 = lambda q: jax.lax.bitcast_convert_type(q, jnp.float32)
    t1h = (f32(a_u & msk) + f32(b_u & msk)
           + d2 * w1l_ref[:, 0:64] + be1_ref[:, 0:64])
    t1l = (f32(a_u << 16) + f32(b_u << 16)
           + d2 * w1l_ref[:, 64:128] + be1_ref[:, 64:128])
    uh = _silu(t1h)
    ul = _silu(t1l)
    m_ij = _silu(jnp.dot(uh, we2_ref[0:64], preferred_element_type=jnp.float32)
                 + jnp.dot(ul, we2_ref[64:128], preferred_element_type=jnp.float32)
                 + be2_ref[...])

    k1 = curv[:, 0:1]
    k2 = curv[:, 1:2]
    hid = (jnp.dot(pr, w16_ref[...], preferred_element_type=jnp.float32)
           + ba1_ref[...])
    terms = (jnp.abs(c1), jnp.abs(c2), c1 * k1, c2 * k2, dist)
    for kk, t in zip((0, 1, 2, 3, 8), terms):
        hid = hid + t * wa1_ref[kk:kk + 1, :]
    hid = _silu(hid)
    w_an = _sigmoid(jnp.sum(hid * wa2_ref[...], axis=1, keepdims=True)
                    + ba2_ref[...]) + 0.5

    m_w = m_ij * w_an

    ch = _silu(jnp.dot(m_w, wh_ref[...], preferred_element_type=jnp.float32)
               + bh_ref[...])
    al = jnp.tanh(jnp.dot(ch, wsel_ref[...],
                          preferred_element_type=jnp.float32)[:, 0:3])
    alpha_b = al[:, 0:1]
    alpha1 = al[:, 1:2]
    alpha2 = al[:, 2:3]

    inv_dn = 1.0 / (dist + 1.0)
    cd = ((alpha_b * inv_dn) * diff + (alpha1 * c1) * pd1r
          + (alpha2 * c2) * pd2r)

    mw_ref[...] = m_w
    cd_ref[...] = jnp.concatenate(
        [cd, jnp.zeros((cd.shape[0], 5), jnp.float32)], axis=1)


def _edge_call(ar, bc, prg, pcg, w1l, be1, we2, be2, wh, bh,
               wsel, wa1, w16, ba1, wa2r, ba2):
    full = lambda shape: pl.BlockSpec(shape, lambda i: tuple(0 for _ in shape))
    return pl.pallas_call(
        _edge_body,
        grid=(_EC // _EB,),
        in_specs=[
            pl.BlockSpec((_EB, 64), lambda i: (i, 0)),
            pl.BlockSpec((_EB, 64), lambda i: (i, 0)),
            pl.BlockSpec((_EB, 16), lambda i: (i, 0)),
            pl.BlockSpec((_EB, 16), lambda i: (i, 0)),
            full((1, D)), full((1, D)), full((D, D)), full((1, D)),
            full((D, 256)), full((1, 256)),
            full((256, D)),
            full((9, 32)), full((16, 32)), full((1, 32)), full((1, 32)),
            full((1, 1)),
        ],
        out_specs=[
            pl.BlockSpec((_EB, D), lambda i: (i, 0)),
            pl.BlockSpec((_EB, 8), lambda i: (i, 0)),
        ],
        out_shape=[
            jax.ShapeDtypeStruct((_EC, D), jnp.float32),
            jax.ShapeDtypeStruct((_EC, 8), jnp.float32),
        ],
    )(ar, bc, prg, pcg, w1l, be1, we2, be2, wh, bh,
      wsel, wa1, w16, ba1, wa2r, ba2)


def _sc_scatter(mws, cds, col, z128, z8):
    mesh = plsc.VectorSubcoreMesh(core_axis_name="c", subcore_axis_name="s")
    rows = _VP // 16

    @functools.partial(
        pl.kernel,
        mesh=mesh,
        compiler_params=pltpu.CompilerParams(use_tc_tiling_on_sc=False),
        out_type=[
            jax.ShapeDtypeStruct((2 * _VP, D), jnp.float32),
            jax.ShapeDtypeStruct((2 * _VP, 8), jnp.float32),
        ],
        scratch_types=[
            pltpu.VMEM((_C,), jnp.int32),
            pltpu.VMEM((_C, D), jnp.float32),
            pltpu.VMEM((_C, 8), jnp.float32),
            pltpu.VMEM_SHARED((_VP, D), jnp.float32),
            pltpu.VMEM_SHARED((_VP, 8), jnp.float32),
            pltpu.SemaphoreType.DMA,
            pltpu.SemaphoreType.DMA,
        ],
    )
    def k(mw0, mw1, mw2, mw3, mw4, cd0, cd1, cd2, cd3, cd4,
          col_hbm, z128_hbm, z8_hbm, out1_hbm, out2_hbm,
          idxc, bufm, bufc, acc128, acc8, sem1, sem2):
        mw_hbms = (mw0, mw1, mw2, mw3, mw4)
        cd_hbms = (cd0, cd1, cd2, cd3, cd4)
        c = lax.axis_index("c")
        s = lax.axis_index("s")
        w = s * 2 + c

        pltpu.sync_copy(z128_hbm.at[pl.ds(s * rows, rows)],
                        acc128.at[pl.ds(s * rows, rows)])
        pltpu.sync_copy(z8_hbm.at[pl.ds(s * rows, rows)],
                        acc8.at[pl.ds(s * rows, rows)])
        plsc.subcore_barrier()

        for ci in range(_NP):
            mw_hbm = mw_hbms[ci]
            cd_hbm = cd_hbms[ci]

            def body(i, carry):
                loc = w * _EWC + i * _C
                pltpu.sync_copy(col_hbm.at[pl.ds(ci * _EC + loc, _C)], idxc)
                cm = pltpu.async_copy(mw_hbm.at[pl.ds(loc, _C)], bufm, sem1)
                cc = pltpu.async_copy(cd_hbm.at[pl.ds(loc, _C)], bufc, sem2)
                cm.wait()
                cc.wait()
                pltpu.sync_copy(bufm, acc128.at[idxc], add=True)
                pltpu.sync_copy(bufc, acc8.at[idxc], add=True)
                return carry

            lax.fori_loop(0, _NCHC, body, 0)
        plsc.subcore_barrier()

        pltpu.sync_copy(acc128.at[pl.ds(s * rows, rows)],
                        out1_hbm.at[pl.ds(c * _VP + s * rows, rows)])
        pltpu.sync_copy(acc8.at[pl.ds(s * rows, rows)],
                        out2_hbm.at[pl.ds(c * _VP + s * rows, rows)])

    return k(*mws, *cds, col, z128, z8)


def _node_body(h_ref, x_ref, s0_ref, s1_ref, c0_ref, c1_ref,
               wn1a_ref, wn1b_ref, bn1_ref, wn2_ref, bn2_ref,
               g_ref, b_ref, hout_ref, xout_ref):
    hb = h_ref[...]
    msg = s0_ref[...] + s1_ref[...]
    u = _silu(jnp.dot(hb, wn1a_ref[...], preferred_element_type=jnp.float32)
              + jnp.dot(msg, wn1b_ref[...], preferred_element_type=jnp.float32)
              + bn1_ref[...])
    hupd = jnp.dot(u, wn2_ref[...], preferred_element_type=jnp.float32) + bn2_ref[...]
    hh = hb + hupd
    mu = jnp.mean(hh, axis=1, keepdims=True)
    var = jnp.mean((hh - mu) ** 2, axis=1, keepdims=True)
    hout_ref[...] = (hh - mu) * lax.rsqrt(var + 1e-5) * g_ref[...] + b_ref[...]
    xout_ref[...] = x_ref[...] + c0_ref[:, 0:3] + c1_ref[:, 0:3]


def _node_call(h, x, s0, s1, c0, c1, wn1a, wn1b, bn1, wn2, bn2, g_ln, b_ln):
    full = lambda shape: pl.BlockSpec(shape, lambda i: tuple(0 for _ in shape))
    return pl.pallas_call(
        _node_body,
        grid=(V // _VB,),
        in_specs=[
            pl.BlockSpec((_VB, D), lambda i: (i, 0)),
            pl.BlockSpec((_VB, 3), lambda i: (i, 0)),
            pl.BlockSpec((_VB, D), lambda i: (i, 0)),
            pl.BlockSpec((_VB, D), lambda i: (i, 0)),
            pl.BlockSpec((_VB, 8), lambda i: (i, 0)),
            pl.BlockSpec((_VB, 8), lambda i: (i, 0)),
            full((D, D)), full((D, D)), full((1, D)),
            full((D, D)), full((1, D)), full((1, D)), full((1, D)),
        ],
        out_specs=[
            pl.BlockSpec((_VB, D), lambda i: (i, 0)),
            pl.BlockSpec((_VB, 3), lambda i: (i, 0)),
        ],
        out_shape=[
            jax.ShapeDtypeStruct((V, D), jnp.float32),
            jax.ShapeDtypeStruct((V, 3), jnp.float32),
        ],
    )(h, x, s0, s1, c0, c1, wn1a, wn1b, bn1, wn2, bn2, g_ln, b_ln)


def kernel(h, x, edge_index, curvature, pd1, pd2, We1, be1, We2, be2,
           Wcb1, bcb1, Wcb2, Wc11, bc11, Wc12, Wc21, bc21, Wc22,
           Wn1, bn1, Wn2, bn2, g_ln, b_ln, Wa1, ba1, Wa2, ba2):
    row = edge_index[0]
    col = edge_index[1]

    w1a = We1[0:D]
    w1b = We1[D:2 * D]
    w1l = We1[2 * D].reshape(1, D)

    pad1 = jnp.zeros((V, 1), jnp.float32)
    P = jnp.concatenate([x, pad1, pd1, pad1, pd2, pad1, curvature], axis=1)

    A, B = _prep_call(h, w1a, w1b)

    wh = jnp.concatenate([Wcb1, Wc11, Wc21], axis=1)
    bh = jnp.concatenate([bcb1, bc11, bc21]).reshape(1, 256)
    w16 = jnp.zeros((16, 32), jnp.float32).at[12:16].set(Wa1[4:8])
    wsel = (jnp.zeros((256, D), jnp.float32)
            .at[0:128, 0].set(Wcb2[:, 0])
            .at[128:192, 1].set(Wc12[:, 0])
            .at[192:256, 2].set(Wc22[:, 0]))

    mws, cds = [], []
    for ci in range(_NP):
        ar, bc, prg, pcg = _sc_gather(A, B, P, row, col, ci * _EC)
        mw, cd8 = _edge_call(
            ar, bc, prg, pcg,
            w1l, be1.reshape(1, D), We2, be2.reshape(1, D), wh, bh,
            wsel, Wa1, w16, ba1.reshape(1, 32), Wa2.reshape(1, 32),
            ba2.reshape(1, 1))
        mws.append(mw)
        cds.append(cd8)

    z128 = jnp.zeros((_VP, D), jnp.float32)
    z8 = jnp.zeros((_VP, 8), jnp.float32)
    s_part, c_part = _sc_scatter(mws, cds, col, z128, z8)

    h_out, x_new = _node_call(
        h, x, s_part[:V], s_part[_VP:_VP + V],
        c_part[:V], c_part[_VP:_VP + V],
        Wn1[0:D], Wn1[D:2 * D], bn1.reshape(1, D), Wn2, bn2.reshape(1, D),
        g_ln.reshape(1, D), b_ln.reshape(1, D))
    return (h_out, x_new)
